# Initial kernel scaffold; baseline (speedup 1.0000x reference)
import jax
import jax.numpy as jnp
from jax.experimental import pallas as pl


def kernel(h_local, h_global, enc_local_line, enc_global, addr_from, addr_to, t, W0_from, b0_from, W1_from, b1_from, W2_from, b2_from, W0_to, b0_to, W1_to, b1_to, W2_to, b2_to):
    raise NotImplementedError("write your pallas kernel here")


# speedup vs baseline: 3.2199x; 3.2199x over previous
import functools

import jax
import jax.numpy as jnp
from jax import lax
from jax.experimental import pallas as pl
from jax.experimental.pallas import tpu as pltpu
from jax.experimental.pallas import tpu_sc as plsc

NC = 2
NS = 16
NW = NC * NS
K = 128


def _sc_mesh():
    return plsc.VectorSubcoreMesh(core_axis_name="c", subcore_axis_name="s",
                                  num_cores=NC, num_subcores=NS)


def _make_gather(n_acc, d, e_pad, nch):
    @functools.partial(
        pl.kernel,
        out_type=jax.ShapeDtypeStruct((2, e_pad, d), jnp.float32),
        mesh=_sc_mesh(),
        scratch_types=[
            pltpu.VMEM((nch, K), jnp.int32),
            pltpu.VMEM((K, d), jnp.float32),
            pltpu.VMEM((K, d), jnp.float32),
            pltpu.SemaphoreType.DMA,
            pltpu.SemaphoreType.DMA,
        ],
    )
    def gather_k(hpad_hbm, idx_hbm, out_hbm, idx_v, rows0, rows1, gsem, osem):
        wid = lax.axis_index("s") * NC + lax.axis_index("c")
        ew = nch * K
        for b in range(2):
            pltpu.sync_copy(idx_hbm.at[b, wid], idx_v)

            def body(jj, carry):
                j0 = 2 * jj
                g0 = pltpu.async_copy(hpad_hbm.at[idx_v.at[j0]], rows0, gsem)
                g1 = pltpu.async_copy(hpad_hbm.at[idx_v.at[j0 + 1]], rows1, gsem)
                g0.wait()
                o0 = pltpu.async_copy(
                    rows0, out_hbm.at[b, pl.ds(wid * ew + j0 * K, K)], osem)
                g1.wait()
                o1 = pltpu.async_copy(
                    rows1, out_hbm.at[b, pl.ds(wid * ew + (j0 + 1) * K, K)], osem)
                o0.wait()
                o1.wait()
                return carry

            lax.fori_loop(0, nch // 2, body, 0)

    return gather_k


def _make_scatter(n, n_acc, d, e_pad, nch):
    @functools.partial(
        pl.kernel,
        out_type=jax.ShapeDtypeStruct((2, n_acc, d), jnp.float32),
        mesh=_sc_mesh(),
        scratch_types=[
            pltpu.VMEM((nch, K), jnp.int32),
            pltpu.VMEM((K, d), jnp.float32),
            pltpu.VMEM((K, d), jnp.float32),
            pltpu.VMEM_SHARED((n_acc, d), jnp.float32),
            pltpu.SemaphoreType.DMA,
        ],
    )
    def scatter_k(r_hbm, idx_hbm, zeros_hbm, out_hbm, idx_v, rows0, rows1, acc, gsem):
        c = lax.axis_index("c")
        s = lax.axis_index("s")
        wid = s * NC + c
        ew = nch * K
        rz = n_acc // NS
        pltpu.sync_copy(zeros_hbm.at[pl.ds(s * rz, rz)], acc.at[pl.ds(s * rz, rz)])
        plsc.subcore_barrier()
        for b in range(2):
            pltpu.sync_copy(idx_hbm.at[b, wid], idx_v)

            def body(jj, carry):
                j0 = 2 * jj
                g0 = pltpu.async_copy(r_hbm.at[b, pl.ds(wid * ew + j0 * K, K)],
                                      rows0, gsem)
                g1 = pltpu.async_copy(r_hbm.at[b, pl.ds(wid * ew + (j0 + 1) * K, K)],
                                      rows1, gsem)
                g0.wait()
                pltpu.sync_copy(rows0, acc.at[idx_v.at[j0]], add=True)
                g1.wait()
                pltpu.sync_copy(rows1, acc.at[idx_v.at[j0 + 1]], add=True)
                return carry

            lax.fori_loop(0, nch // 2, body, 0)
        plsc.subcore_barrier()
        ro = n_acc // NS
        pltpu.sync_copy(acc.at[pl.ds(s * ro, ro)], out_hbm.at[c, pl.ds(s * ro, ro)])

    return scatter_k


def _mlp_block(gath_ref, enc_ref, hg_ref, eg_ref,
               w0a_ref, w0b_ref, w0g_ref, w0e_ref, tb0_ref,
               w1_ref, b1_ref, w2_ref, b2_ref, r_ref):
    enc = enc_ref[...]
    f32 = jnp.float32
    for b in range(2):
        c0 = (jnp.dot(hg_ref[...], w0g_ref[b], preferred_element_type=f32)
              + jnp.dot(eg_ref[...], w0e_ref[b], preferred_element_type=f32)
              + tb0_ref[b])
        x = jnp.tanh(jnp.dot(gath_ref[b], w0a_ref[b], preferred_element_type=f32)
                     + jnp.dot(enc, w0b_ref[b], preferred_element_type=f32)
                     + c0)
        x = jnp.tanh(jnp.dot(x, w1_ref[b], preferred_element_type=f32) + b1_ref[b])
        r_ref[b] = jnp.tanh(jnp.dot(x, w2_ref[b], preferred_element_type=f32)
                            + b2_ref[b])


def _final_block(p_ref, o_ref):
    o_ref[...] = jnp.tanh(p_ref[0] + p_ref[1])


def kernel(h_local, h_global, enc_local_line, enc_global, addr_from, addr_to, t,
           W0_from, b0_from, W1_from, b1_from, W2_from, b2_from,
           W0_to, b0_to, W1_to, b1_to, W2_to, b2_to):
    n, d = h_local.shape
    e = addr_from.shape[0]
    h = W1_from.shape[0]

    ew = -(-e // (NW * 2 * K)) * 2 * K
    e_pad = ew * NW
    nch = ew // K
    n_acc = -(-(n + 16) // (NS * 8)) * (NS * 8)

    pad_idx = n + (jnp.arange(e_pad - e, dtype=jnp.int32) % 16)
    idx = jnp.stack([addr_from.astype(jnp.int32), addr_to.astype(jnp.int32)])
    idx = jnp.concatenate([idx, jnp.stack([pad_idx, pad_idx])], axis=1)
    idx = idx.reshape(2, NW, nch, K)
    hpad = jnp.concatenate([h_local, jnp.zeros((n_acc - n, d), jnp.float32)])
    zeros_acc = jnp.zeros((n_acc, d), jnp.float32)

    w0a = jnp.stack([W0_from[0 * d:1 * d], W0_to[0 * d:1 * d]])
    w0g = jnp.stack([W0_from[1 * d:2 * d], W0_to[1 * d:2 * d]])
    w0b = jnp.stack([W0_from[2 * d:3 * d], W0_to[2 * d:3 * d]])
    w0e = jnp.stack([W0_from[3 * d:4 * d], W0_to[3 * d:4 * d]])
    tb0 = jnp.stack([t * W0_from[4 * d] + b0_from,
                     t * W0_to[4 * d] + b0_to]).reshape(2, 1, h)
    w1 = jnp.stack([W1_from, W1_to])
    b1 = jnp.stack([b1_from, b1_to]).reshape(2, 1, h)
    w2 = jnp.stack([W2_from, W2_to])
    b2 = jnp.stack([b2_from, b2_to]).reshape(2, 1, d)

    gath = _make_gather(n_acc, d, e_pad, nch)(hpad, idx)

    be = 1280
    nb = e // be
    wspec = lambda shp: pl.BlockSpec(shp, lambda i: (0,) * len(shp))
    r = pl.pallas_call(
        _mlp_block,
        grid=(nb,),
        in_specs=[
            pl.BlockSpec((2, be, d), lambda i: (0, i, 0)),
            pl.BlockSpec((be, d), lambda i: (i, 0)),
            wspec((1, d)), wspec((1, d)),
            wspec((2, d, h)), wspec((2, d, h)),
            wspec((2, d, h)), wspec((2, d, h)),
            wspec((2, 1, h)),
            wspec((2, h, h)), wspec((2, 1, h)),
            wspec((2, h, d)), wspec((2, 1, d)),
        ],
        out_specs=pl.BlockSpec((2, be, d), lambda i: (0, i, 0)),
        out_shape=jax.ShapeDtypeStruct((2, e_pad, d), jnp.float32),
    )(gath, enc_local_line, h_global, enc_global,
      w0a, w0b, w0g, w0e, tb0, w1, b1, w2, b2)

    partial = _make_scatter(n, n_acc, d, e_pad, nch)(r, idx, zeros_acc)

    bn = 1000
    out = pl.pallas_call(
        _final_block,
        grid=(n // bn,),
        in_specs=[pl.BlockSpec((2, bn, d), lambda i: (0, i, 0))],
        out_specs=pl.BlockSpec((bn, d), lambda i: (i, 0)),
        out_shape=jax.ShapeDtypeStruct((n, d), jnp.float32),
    )(partial)
    return out
